# baseline (speedup 1.0000x reference)
import functools

import jax
import jax.numpy as jnp
from jax import lax
from jax.experimental import pallas as pl
from jax.experimental.pallas import tpu as pltpu
from jax.experimental.pallas import tpu_sc as plsc

B = 16384
F = 100
NC = 2
NS = 16
NW = NC * NS
RPW = B // NW
L = 16
NCHUNK = 5
CF = F // NCHUNK
CVALS = CF * RPW

_mesh = plsc.VectorSubcoreMesh(core_axis_name="c", subcore_axis_name="s")

_scratch = (
    [pltpu.VMEM((CVALS,), jnp.int32) for _ in range(NCHUNK)]
    + [pltpu.VMEM((CVALS,), jnp.float32) for _ in range(NCHUNK)]
    + [
        pltpu.VMEM((RPW,), jnp.float32),
        pltpu.SemaphoreType.DMA,
        pltpu.SemaphoreType.DMA,
    ]
)


@functools.partial(
    pl.kernel,
    mesh=_mesh,
    out_type=jax.ShapeDtypeStruct((B,), jnp.float32),
    scratch_types=_scratch,
)
def _lookup_sum_sigmoid(xt_hbm, table_hbm, out_hbm, *refs):
    idx_refs = refs[:NCHUNK]
    val_refs = refs[NCHUNK:2 * NCHUNK]
    out_v, idx_sem, gat_sem = refs[2 * NCHUNK:]
    table_1d = table_hbm.at[0]

    wid = lax.axis_index("s") * NC + lax.axis_index("c")
    base = wid * RPW

    def stage(k):
        def body(j, carry):
            pltpu.async_copy(
                xt_hbm.at[k * CF + j, pl.ds(base, RPW)],
                idx_refs[k].at[pl.ds(j * RPW, RPW)],
                idx_sem,
            )
            return carry

        lax.fori_loop(0, CF, body, 0)

    def drain_stage(k):
        def body(j, carry):
            pltpu.make_async_copy(
                xt_hbm.at[k * CF + j, pl.ds(base, RPW)],
                idx_refs[k].at[pl.ds(j * RPW, RPW)],
                idx_sem,
            ).wait()
            return carry

        lax.fori_loop(0, CF, body, 0)

    def reduce_chunk(k):
        vals = val_refs[k]
        last = k == NCHUNK - 1

        def body(g, carry):
            acc = vals[pl.ds(g * L, L)]
            for j in range(1, CF):
                acc = acc + vals[pl.ds(j * RPW + g * L, L)]
            if k == 0:
                out_v[pl.ds(g * L, L)] = acc
            elif last:
                s = out_v[pl.ds(g * L, L)] + acc
                out_v[pl.ds(g * L, L)] = 1.0 / (1.0 + jnp.exp(-s))
            else:
                out_v[pl.ds(g * L, L)] = out_v[pl.ds(g * L, L)] + acc
            return carry

        lax.fori_loop(0, RPW // L, body, 0)

    DEPTH = 4
    stage(0)
    for k in range(NCHUNK):
        drain_stage(k)
        pltpu.async_copy(table_1d.at[idx_refs[k]], val_refs[k], gat_sem)
        if k + 1 < NCHUNK:
            stage(k + 1)
        if k >= DEPTH - 1:
            j = k - (DEPTH - 1)
            pltpu.make_async_copy(
                table_1d.at[idx_refs[j]], val_refs[j], gat_sem
            ).wait()
            reduce_chunk(j)
    for j in range(NCHUNK - DEPTH + 1, NCHUNK):
        pltpu.make_async_copy(
            table_1d.at[idx_refs[j]], val_refs[j], gat_sem
        ).wait()
        reduce_chunk(j)

    pltpu.sync_copy(out_v, out_hbm.at[pl.ds(base, RPW)])


def kernel(x, table):
    xt = x.astype(jnp.int32).T
    out = _lookup_sum_sigmoid(xt, table.reshape(1, -1))
    return out.reshape(B, 1)


# speedup vs baseline: 1.0099x; 1.0099x over previous
import functools

import jax
import jax.numpy as jnp
from jax import lax
from jax.experimental import pallas as pl
from jax.experimental.pallas import tpu as pltpu
from jax.experimental.pallas import tpu_sc as plsc

B = 16384
F = 100
NC = 2
NS = 16
NW = NC * NS
RPW = B // NW
L = 16
NCHUNK = 10
CF = F // NCHUNK
CVALS = CF * RPW

_mesh = plsc.VectorSubcoreMesh(core_axis_name="c", subcore_axis_name="s")

_scratch = (
    [pltpu.VMEM((CVALS,), jnp.int32) for _ in range(NCHUNK)]
    + [pltpu.VMEM((CVALS,), jnp.float32) for _ in range(NCHUNK)]
    + [
        pltpu.VMEM((RPW,), jnp.float32),
        pltpu.SemaphoreType.DMA,
        pltpu.SemaphoreType.DMA,
    ]
)


@functools.partial(
    pl.kernel,
    mesh=_mesh,
    out_type=jax.ShapeDtypeStruct((B,), jnp.float32),
    scratch_types=_scratch,
)
def _lookup_sum_sigmoid(xt_hbm, table_hbm, out_hbm, *refs):
    idx_refs = refs[:NCHUNK]
    val_refs = refs[NCHUNK:2 * NCHUNK]
    out_v, idx_sem, gat_sem = refs[2 * NCHUNK:]
    table_1d = table_hbm.at[0]

    wid = lax.axis_index("s") * NC + lax.axis_index("c")
    base = wid * RPW

    def stage(k):
        def body(j, carry):
            pltpu.async_copy(
                xt_hbm.at[k * CF + j, pl.ds(base, RPW)],
                idx_refs[k].at[pl.ds(j * RPW, RPW)],
                idx_sem,
            )
            return carry

        lax.fori_loop(0, CF, body, 0)

    def drain_stage(k):
        def body(j, carry):
            pltpu.make_async_copy(
                xt_hbm.at[k * CF + j, pl.ds(base, RPW)],
                idx_refs[k].at[pl.ds(j * RPW, RPW)],
                idx_sem,
            ).wait()
            return carry

        lax.fori_loop(0, CF, body, 0)

    def reduce_chunk(k):
        vals = val_refs[k]
        last = k == NCHUNK - 1

        def body(g, carry):
            acc = vals[pl.ds(g * L, L)]
            for j in range(1, CF):
                acc = acc + vals[pl.ds(j * RPW + g * L, L)]
            if k == 0:
                out_v[pl.ds(g * L, L)] = acc
            elif last:
                s = out_v[pl.ds(g * L, L)] + acc
                out_v[pl.ds(g * L, L)] = 1.0 / (1.0 + jnp.exp(-s))
            else:
                out_v[pl.ds(g * L, L)] = out_v[pl.ds(g * L, L)] + acc
            return carry

        lax.fori_loop(0, RPW // L, body, 0)

    DEPTH = 4
    stage(0)
    for k in range(NCHUNK):
        drain_stage(k)
        pltpu.async_copy(table_1d.at[idx_refs[k]], val_refs[k], gat_sem)
        if k + 1 < NCHUNK:
            stage(k + 1)
        if k >= DEPTH - 1:
            j = k - (DEPTH - 1)
            pltpu.make_async_copy(
                table_1d.at[idx_refs[j]], val_refs[j], gat_sem
            ).wait()
            reduce_chunk(j)
    for j in range(NCHUNK - DEPTH + 1, NCHUNK):
        pltpu.make_async_copy(
            table_1d.at[idx_refs[j]], val_refs[j], gat_sem
        ).wait()
        reduce_chunk(j)

    pltpu.sync_copy(out_v, out_hbm.at[pl.ds(base, RPW)])


def kernel(x, table):
    xt = x.astype(jnp.int32).T
    out = _lookup_sum_sigmoid(xt, table.reshape(1, -1))
    return out.reshape(B, 1)
